# baseline (speedup 1.0000x reference)
import functools

import jax
import jax.numpy as jnp
from jax import lax
from jax.experimental import pallas as pl
from jax.experimental.pallas import tpu as pltpu
from jax.experimental.pallas import tpu_sc as plsc

F32 = jnp.float32

ROW = 192
MROW = 176
CROW = 16
CH = 80
NB = 1000
EB = 2000
NC = 2
NS = 16


def _table_body(sf_ref, wgh_ref, p_ref, stats_ref):
    sf = sf_ref[...]
    p_ref[...] = jnp.dot(sf, wgh_ref[...], preferred_element_type=F32)
    stats_ref[:, 0:1] = jnp.sum(sf, axis=1, keepdims=True)
    stats_ref[:, 1:2] = jnp.sum(sf * sf, axis=1, keepdims=True)


def _edge_body(a_ref, b_ref, whD_ref, wh0D_ref, gsum_ref, wgs1_ref,
               ones17_ref, rs_ref, c1_ref, g1w3_ref, g1b3_ref, wu1D_ref,
               out_ref):
    a = a_ref[...]
    b = b_ref[...]
    p = a[:, 0:128]
    s1 = a[:, 179:180]
    s2 = a[:, 180:181]

    vall = a[:, 128:176]
    d3 = a[:, 176:179] - b[:, 0:3]
    vh = (jnp.dot(vall, whD_ref[...], preferred_element_type=F32)
          + jnp.dot(d3, wh0D_ref[...], preferred_element_type=F32))
    sq = vh * vh
    sh_sq = jnp.dot(sq, gsum_ref[...], preferred_element_type=F32) + 1e-8
    sh = jnp.sqrt(sh_sq)

    inv_d = 1.0 / 145.0
    t_ext = jnp.dot(sh, wgs1_ref[...], preferred_element_type=F32)
    sqsum = jnp.dot(sh_sq, ones17_ref[...], preferred_element_type=F32)
    m = (s1 + t_ext[:, 128:129]) * inv_d
    var = (s2 + sqsum) * inv_d - m * m
    inv_std = lax.rsqrt(var + 1e-5)

    y = (p + t_ext[:, 0:128] - m * rs_ref[...]) * inv_std + c1_ref[...]
    feats = y * jax.nn.sigmoid(y)

    gates3 = jnp.dot(feats, g1w3_ref[...],
                     preferred_element_type=F32) + g1b3_ref[...]
    gv3 = gates3 * jax.nn.sigmoid(gates3)
    vu = jnp.dot(vh, wu1D_ref[...], preferred_element_type=F32)

    out_ref[:, 0:128] = feats
    out_ref[:, 128:176] = gv3 * vu


def _node_body(s0_ref, vf_ref, acc_ref, lng_ref, lnb_ref, wh2_ref, wu2_ref,
               wg2h_ref, wg2s_ref, rs2_ref, c2_ref, g2w_ref, g2b_ref,
               s_out_ref, v_out_ref):
    acc = acc_ref[...]
    p0 = acc[0]
    p1 = acc[1]
    msg_s = jnp.concatenate([p0[:, 0:96], p1[:, 16:48]], axis=1)
    s = s0_ref[...] + msg_s
    vf = vf_ref[...]
    v = [vf[:, 16 * c:16 * c + 16] + p1[:, 48 + 16 * c:64 + 16 * c]
         for c in range(3)]

    lng = lng_ref[...]
    lnb = lnb_ref[...]
    mu = jnp.mean(s, axis=1, keepdims=True)
    var = jnp.mean(s * s, axis=1, keepdims=True) - mu * mu
    s_ln = (s - mu) * lax.rsqrt(var + 1e-5) * lng + lnb
    vq = v[0] * v[0] + v[1] * v[1] + v[2] * v[2]
    vn = jnp.sqrt(jnp.mean(vq, axis=1, keepdims=True) + 1e-8)
    inv_vn = 1.0 / vn
    vhat = [vc * inv_vn for vc in v]

    wh2 = wh2_ref[...]
    vh2 = [jnp.dot(vc, wh2, preferred_element_type=F32) for vc in vhat]
    sh_sq = vh2[0] * vh2[0] + vh2[1] * vh2[1] + vh2[2] * vh2[2] + 1e-8
    sh = jnp.sqrt(sh_sq)
    inv_d = 1.0 / 145.0
    m2 = (jnp.sum(s_ln, axis=1, keepdims=True)
          + jnp.sum(sh, axis=1, keepdims=True)) * inv_d
    var2 = (jnp.sum(s_ln * s_ln, axis=1, keepdims=True)
            + jnp.sum(sh_sq, axis=1, keepdims=True)) * inv_d - m2 * m2
    inv_std2 = lax.rsqrt(var2 + 1e-5)
    t2 = (jnp.dot(s_ln, wg2h_ref[...], preferred_element_type=F32)
          + jnp.dot(sh, wg2s_ref[...], preferred_element_type=F32))
    y2 = (t2 - m2 * rs2_ref[...]) * inv_std2 + c2_ref[...]
    feats2 = y2 * jax.nn.sigmoid(y2)
    gates2 = jnp.dot(feats2, g2w_ref[...], preferred_element_type=F32) + g2b_ref[...]
    gv2 = gates2 * jax.nn.sigmoid(gates2)

    wu2 = wu2_ref[...]
    s_new = s_ln + feats2
    v_new = []
    for c in range(3):
        vu2 = jnp.dot(vh2[c], wu2, preferred_element_type=F32)
        vres = gv2 * vu2
        v_new.append(vhat[c] + vres[:, 0:16])

    mu3 = jnp.mean(s_new, axis=1, keepdims=True)
    var3 = jnp.mean(s_new * s_new, axis=1, keepdims=True) - mu3 * mu3
    s_out_ref[...] = (s_new - mu3) * lax.rsqrt(var3 + 1e-5) * lng + lnb
    vq3 = v_new[0] * v_new[0] + v_new[1] * v_new[1] + v_new[2] * v_new[2]
    inv_vn3 = 1.0 / jnp.sqrt(jnp.mean(vq3, axis=1, keepdims=True) + 1e-8)
    for c in range(3):
        v_out_ref[:, 16 * c:16 * c + 16] = v_new[c] * inv_vn3


NBUF = 5


def _make_gather(n_edges):
    mesh = plsc.VectorSubcoreMesh(core_axis_name="c", subcore_axis_name="s",
                                  num_cores=NC, num_subcores=NS)
    per_w = n_edges // (NC * NS)
    n_ch = per_w // CH
    assert n_ch % NBUF == 0

    @functools.partial(
        pl.kernel,
        out_type=(jax.ShapeDtypeStruct((n_edges, ROW), F32),
                  jax.ShapeDtypeStruct((n_edges, CROW), F32)),
        mesh=mesh,
        scratch_types=(
            [pltpu.VMEM((CH,), jnp.int32) for _ in range(NBUF)]
            + [pltpu.VMEM((CH,), jnp.int32) for _ in range(NBUF)]
            + [pltpu.VMEM((CH, ROW), F32) for _ in range(NBUF)]
            + [pltpu.VMEM((CH, CROW), F32) for _ in range(NBUF)]
            + [pltpu.SemaphoreType.DMA for _ in range(4 * NBUF)]
        ),
        compiler_params=pltpu.CompilerParams(use_tc_tiling_on_sc=False),
    )
    def gather_k(tbl, ctbl, src, dst, a_out, b_out, *scr):
        idx_s = scr[0:NBUF]
        idx_d = scr[NBUF:2 * NBUF]
        rows = scr[2 * NBUF:3 * NBUF]
        crows = scr[3 * NBUF:4 * NBUF]
        gsa = scr[4 * NBUF:5 * NBUF]
        gsb = scr[5 * NBUF:6 * NBUF]
        wsa = scr[6 * NBUF:7 * NBUF]
        wsb = scr[7 * NBUF:8 * NBUF]
        wid = lax.axis_index("s") * NC + lax.axis_index("c")
        base0 = pl.multiple_of(wid * per_w, 8)

        for b in range(NBUF):
            base = pl.multiple_of(base0 + b * CH, 8)
            pltpu.sync_copy(src.at[pl.ds(base, CH)], idx_s[b])
            pltpu.sync_copy(dst.at[pl.ds(base, CH)], idx_d[b])
            pltpu.async_copy(tbl.at[idx_s[b]], rows[b], gsa[b])
            pltpu.async_copy(ctbl.at[idx_d[b]], crows[b], gsb[b])

        def body(g, carry):
            for b in range(NBUF):
                base = pl.multiple_of(base0 + (g * NBUF + b) * CH, 8)
                pltpu.make_async_copy(tbl.at[idx_s[b]], rows[b], gsa[b]).wait()
                pltpu.make_async_copy(ctbl.at[idx_d[b]], crows[b], gsb[b]).wait()
                pltpu.async_copy(rows[b], a_out.at[pl.ds(base, CH)], wsa[b])
                pltpu.async_copy(crows[b], b_out.at[pl.ds(base, CH)], wsb[b])

                @pl.when(g < (n_ch // NBUF) - 1)
                def _():
                    nbase = pl.multiple_of(base + NBUF * CH, 8)
                    pltpu.sync_copy(src.at[pl.ds(nbase, CH)], idx_s[b])
                    pltpu.sync_copy(dst.at[pl.ds(nbase, CH)], idx_d[b])
                    pltpu.make_async_copy(
                        rows[b], a_out.at[pl.ds(base, CH)], wsa[b]).wait()
                    pltpu.make_async_copy(
                        crows[b], b_out.at[pl.ds(base, CH)], wsb[b]).wait()
                    pltpu.async_copy(tbl.at[idx_s[b]], rows[b], gsa[b])
                    pltpu.async_copy(ctbl.at[idx_d[b]], crows[b], gsb[b])
            return carry

        lax.fori_loop(0, n_ch // NBUF, body, 0)
        last0 = pl.multiple_of(base0 + (n_ch - NBUF) * CH, 8)
        for b in range(NBUF):
            base = pl.multiple_of(last0 + b * CH, 8)
            pltpu.make_async_copy(rows[b], a_out.at[pl.ds(base, CH)], wsa[b]).wait()
            pltpu.make_async_copy(crows[b], b_out.at[pl.ds(base, CH)], wsb[b]).wait()

    return gather_k


SCW = 96


def _make_scatter(n_nodes, n_edges):
    mesh = plsc.VectorSubcoreMesh(core_axis_name="c", subcore_axis_name="s",
                                  num_cores=NC, num_subcores=NS)
    rows_pt = n_nodes // NS
    per_tile = n_edges // NS
    n_ch = per_tile // CH
    assert n_ch % NBUF == 0

    @functools.partial(
        pl.kernel,
        out_type=jax.ShapeDtypeStruct((NC, n_nodes, SCW), F32),
        mesh=mesh,
        scratch_types=(
            [pltpu.VMEM_SHARED((n_nodes, SCW), F32)]
            + [pltpu.VMEM((CH, SCW), F32) for _ in range(NBUF)]
            + [pltpu.VMEM((CH,), jnp.int32) for _ in range(NBUF)]
            + [pltpu.SemaphoreType.DMA for _ in range(2 * NBUF)]
        ),
        compiler_params=pltpu.CompilerParams(use_tc_tiling_on_sc=False),
    )
    def scatter_k(m, dstv, zeros, out, accum, *scr):
        mbuf = scr[0:NBUF]
        idxb = scr[NBUF:2 * NBUF]
        msem = scr[2 * NBUF:3 * NBUF]
        ssem = scr[3 * NBUF:4 * NBUF]
        cid = lax.axis_index("c")
        sid = lax.axis_index("s")
        col0 = cid * (MROW - SCW)
        pltpu.sync_copy(zeros, accum.at[pl.ds(sid * rows_pt, rows_pt)])
        plsc.subcore_barrier()

        base0 = pl.multiple_of(sid * per_tile, 8)

        for b in range(NBUF):
            base = pl.multiple_of(base0 + b * CH, 8)
            pltpu.sync_copy(dstv.at[pl.ds(base, CH)], idxb[b])
            pltpu.async_copy(m.at[pl.ds(base, CH), pl.ds(col0, SCW)],
                             mbuf[b], msem[b])

        def body(g, carry):
            for b in range(NBUF):
                base = pl.multiple_of(base0 + (g * NBUF + b) * CH, 8)
                pltpu.make_async_copy(m.at[pl.ds(base, CH), pl.ds(col0, SCW)],
                                      mbuf[b], msem[b]).wait()
                pltpu.async_copy(mbuf[b], accum.at[idxb[b]], ssem[b], add=True)

                @pl.when(g < (n_ch // NBUF) - 1)
                def _():
                    nbase = pl.multiple_of(base + NBUF * CH, 8)
                    pltpu.make_async_copy(mbuf[b], accum.at[idxb[b]],
                                          ssem[b]).wait()
                    pltpu.sync_copy(dstv.at[pl.ds(nbase, CH)], idxb[b])
                    pltpu.async_copy(m.at[pl.ds(nbase, CH), pl.ds(col0, SCW)],
                                     mbuf[b], msem[b])
            return carry

        lax.fori_loop(0, n_ch // NBUF, body, 0)
        for b in range(NBUF):
            pltpu.make_async_copy(mbuf[b], accum.at[idxb[b]], ssem[b]).wait()
        plsc.subcore_barrier()
        pltpu.sync_copy(accum.at[pl.ds(sid * rows_pt, rows_pt)],
                        out.at[cid, pl.ds(sid * rows_pt, rows_pt)])

    return scatter_k


def kernel(scalar_feat, coord_feat, vec_feat, edge_index,
           Wh1, Wu1, ln1_g, ln1_b, lin1_w, lin1_b, gate1_w, gate1_b,
           Wh2, Wu2, ln2_g, ln2_b, lin2_w, lin2_b, gate2_w, gate2_b,
           lnorm_g, lnorm_b):
    n, s_dim = scalar_feat.shape
    e = edge_index.shape[1]
    v_dim = vec_feat.shape[1]
    assert s_dim == 128 and v_dim == 16
    assert n % NB == 0 and e % EB == 0 and e % (NC * NS * CH) == 0
    assert n % NS == 0

    wg1 = lin1_w * ln1_g[None, :]
    wg1h = wg1[:, :s_dim].T
    wg1s = wg1[:, s_dim:].T
    rs1 = jnp.sum(wg1, axis=1)[None, :]
    c1 = (ln1_b @ lin1_w.T + lin1_b)[None, :]
    wh10 = Wh1[0:1, :]
    wh1v = Wh1[1:, :]
    h1 = Wh1.shape[1]
    zeros_b = jnp.zeros((v_dim, h1), F32)
    whD = jnp.block([[wh1v, zeros_b, zeros_b],
                     [zeros_b, wh1v, zeros_b],
                     [zeros_b, zeros_b, wh1v]])
    z1h = jnp.zeros((1, h1), F32)
    wh0D = jnp.block([[wh10, z1h, z1h],
                      [z1h, wh10, z1h],
                      [z1h, z1h, wh10]])
    eye_h = jnp.eye(h1, dtype=F32)
    gsum = jnp.concatenate([eye_h, eye_h, eye_h], axis=0)
    wgs1 = jnp.concatenate([wg1s, jnp.ones((h1, 1), F32)], axis=1)
    ones17 = jnp.ones((h1, 1), F32)
    g1w3 = jnp.tile(gate1_w.T, (1, 3))
    g1b3 = jnp.tile(gate1_b[None, :], (1, 3))
    zu = jnp.zeros((h1, v_dim), F32)
    wu1D = jnp.block([[Wu1, zu, zu],
                      [zu, Wu1, zu],
                      [zu, zu, Wu1]])

    wg2 = lin2_w * ln2_g[None, :]
    wg2h = wg2[:, :s_dim].T
    wg2s = wg2[:, s_dim:].T
    rs2 = jnp.sum(wg2, axis=1)[None, :]
    c2 = (ln2_b @ lin2_w.T + lin2_b)[None, :]
    g2w = gate2_w.T
    g2b = gate2_b[None, :]
    lng = lnorm_g[None, :]
    lnb = lnorm_b[None, :]

    vflat = vec_feat.transpose(0, 2, 1).reshape(n, 3 * v_dim)
    src = edge_index[0]
    dst = edge_index[1]

    grid_n = n // NB
    p, stats = pl.pallas_call(
        _table_body,
        grid=(grid_n,),
        in_specs=[
            pl.BlockSpec((NB, s_dim), lambda i: (i, 0)),
            pl.BlockSpec((s_dim, s_dim), lambda i: (0, 0)),
        ],
        out_specs=[
            pl.BlockSpec((NB, s_dim), lambda i: (i, 0)),
            pl.BlockSpec((NB, 16), lambda i: (i, 0)),
        ],
        out_shape=[
            jax.ShapeDtypeStruct((n, s_dim), F32),
            jax.ShapeDtypeStruct((n, 16), F32),
        ],
    )(scalar_feat, wg1h)

    tail = jnp.concatenate(
        [coord_feat, stats[:, 0:1], stats[:, 1:2],
         jnp.zeros((n, ROW - 128 - 48 - 5), F32)], axis=1)
    tbl = jnp.concatenate([p, vflat, tail], axis=1)
    ctbl = jnp.concatenate(
        [coord_feat, jnp.zeros((n, CROW - 3), F32)], axis=1)

    a_edges, b_edges = _make_gather(e)(tbl, ctbl, src, dst)

    grid_e = e // EB
    wspec = lambda shape: pl.BlockSpec(shape, lambda i: tuple(0 for _ in shape))
    m_edges = pl.pallas_call(
        _edge_body,
        grid=(grid_e,),
        in_specs=[
            pl.BlockSpec((EB, ROW), lambda i: (i, 0)),
            pl.BlockSpec((EB, CROW), lambda i: (i, 0)),
            wspec((48, 51)), wspec((3, 51)), wspec((51, 17)),
            wspec((17, 129)), wspec((17, 1)), wspec((1, 128)),
            wspec((1, 128)), wspec((128, 48)), wspec((1, 48)),
            wspec((51, 48)),
        ],
        out_specs=pl.BlockSpec((EB, MROW), lambda i: (i, 0)),
        out_shape=jax.ShapeDtypeStruct((e, MROW), F32),
    )(a_edges, b_edges, whD, wh0D, gsum, wgs1, ones17, rs1, c1, g1w3,
      g1b3, wu1D)

    zeros = jnp.zeros((n // NS, SCW), F32)
    partials = _make_scatter(n, e)(m_edges, dst, zeros)

    s_out, vflat_out = pl.pallas_call(
        _node_body,
        grid=(grid_n,),
        in_specs=[
            pl.BlockSpec((NB, s_dim), lambda i: (i, 0)),
            pl.BlockSpec((NB, 48), lambda i: (i, 0)),
            pl.BlockSpec((NC, NB, SCW), lambda i: (0, i, 0)),
            wspec((1, 128)), wspec((1, 128)),
            wspec((16, 17)), wspec((17, 17)),
            wspec((128, 128)), wspec((17, 128)),
            wspec((1, 128)), wspec((1, 128)),
            wspec((128, 17)), wspec((1, 17)),
        ],
        out_specs=[
            pl.BlockSpec((NB, s_dim), lambda i: (i, 0)),
            pl.BlockSpec((NB, 48), lambda i: (i, 0)),
        ],
        out_shape=[
            jax.ShapeDtypeStruct((n, s_dim), F32),
            jax.ShapeDtypeStruct((n, 48), F32),
        ],
    )(scalar_feat, vflat, partials, lng, lnb, Wh2, Wu2,
      wg2h, wg2s, rs2, c2, g2w, g2b)

    v_out = vflat_out.reshape(n, 3, v_dim).transpose(0, 2, 1)
    return s_out, v_out
